# baseline (speedup 1.0000x reference)
import functools

import jax
import jax.numpy as jnp
from jax import lax
from jax.experimental import pallas as pl
from jax.experimental.pallas import tpu as pltpu
from jax.experimental.pallas import tpu_sc as plsc

_P = 64
_A = 16
_NC = 2
_NS = 16
_NW = _NC * _NS
_L = 16
_CHUNK = 5120
_NCHUNK = 2
_PC = _P // _NCHUNK
_QR = 128


def _sc_gather_rows(W, tok, k):
    V = W.shape[1]
    N = tok.shape[0]
    mesh = plsc.VectorSubcoreMesh(
        core_axis_name="c", subcore_axis_name="s",
        num_cores=_NC, num_subcores=_NS)

    @functools.partial(
        pl.kernel,
        out_type=jax.ShapeDtypeStruct((_PC, N), jnp.float32),
        mesh=mesh,
        compiler_params=pltpu.CompilerParams(needs_layout_passes=False),
        scratch_types=[
            pltpu.VMEM((N,), jnp.int32),
            pltpu.VMEM((V,), jnp.float32),
            pltpu.VMEM((_CHUNK,), jnp.float32)
        ],
    )
    def sck(w_hbm, tok_hbm, ut_hbm, tok_v, w_v, out_v):
        wid = lax.axis_index("s") * _NC + lax.axis_index("c")
        pltpu.sync_copy(tok_hbm, tok_v)
        pltpu.sync_copy(w_hbm.at[_PC * k + wid], w_v)

        def chunk_body(c, _):
            base = pl.multiple_of(c * _CHUNK, _CHUNK)

            def g(i, _):
                idx = tok_v[pl.ds(base + i * _L, _L)]
                out_v[pl.ds(i * _L, _L)] = plsc.load_gather(w_v, [idx])
                return 0

            lax.fori_loop(0, _CHUNK // _L, g, 0, unroll=8)
            pltpu.sync_copy(out_v, ut_hbm.at[wid, pl.ds(base, _CHUNK)])
            return 0

        lax.fori_loop(0, N // _CHUNK, chunk_body, 0)

    return sck(W, tok)


def _tc_expand_chunk(UTc, SPc, fT, k, prev):
    N = UTc.shape[1]
    PA = _P * _A

    def body(ut_ref, sp_ref, ft_ref, *rest):
        u_ref, fh_ref, y_ref = rest[-3:]
        q = pl.program_id(0)
        ut = ut_ref[...]
        fh = (ut >= sp_ref[...]).astype(jnp.float32)
        t = fh * ut
        ftv = ft_ref[...]
        tiled = jnp.concatenate([ftv] * (_QR // _A), axis=0)
        rloc = (lax.broadcasted_iota(jnp.int32, (_QR, _PC), 0) + q * _QR) // _A
        ploc = lax.broadcasted_iota(jnp.int32, (_QR, _PC), 1)
        et_q = jnp.where(rloc == ploc, tiled, 0.0)
        y_ref[...] = jnp.dot(et_q, t, preferred_element_type=jnp.float32)
        u_ref[...] = ut
        fh_ref[...] = fh

    grid_len = (_PC * _A) // _QR
    hbm = pl.BlockSpec(memory_space=pltpu.MemorySpace.HBM)

    in_specs = [
        pl.BlockSpec((_PC, N), lambda q: (0, 0)),
        pl.BlockSpec((_PC, 1), lambda q: (k, 0)),
        pl.BlockSpec((_A, _PC), lambda q: (0, 0)),
    ]
    operands = [UTc, SPc, fT[:, _PC * k:_PC * (k + 1)]]
    aliases = {}
    if prev is not None:
        in_specs += [hbm, hbm, hbm]
        operands += list(prev)
        aliases = {3: 0, 4: 1, 5: 2}

    return pl.pallas_call(
        body,
        grid=(grid_len,),
        in_specs=in_specs,
        out_specs=[
            pl.BlockSpec((_PC, N), lambda q: (k, 0)),
            pl.BlockSpec((_PC, N), lambda q: (k, 0)),
            pl.BlockSpec((_QR, N), lambda q: (q + grid_len * k, 0)),
        ],
        out_shape=[
            jax.ShapeDtypeStruct((_P, N), jnp.float32),
            jax.ShapeDtypeStruct((_P, N), jnp.float32),
            jax.ShapeDtypeStruct((PA, N), jnp.float32),
        ],
        input_output_aliases=aliases,
    )(*operands)


def kernel(input_ids, W, f, SP):
    B, T = input_ids.shape
    N = B * T
    PA = _P * _A
    tok = input_ids.reshape(N)
    SPc = SP.reshape(_P, 1)
    fT = f.T

    prev = None
    for k in range(_NCHUNK):
        UTc = _sc_gather_rows(W, tok, k)
        prev = _tc_expand_chunk(UTc, SPc, fT, k, prev)
    U_T, FhT, Y2T = prev

    U = U_T.T
    Fh = FhT.T
    Y = Y2T.reshape(_P, _A, N).transpose(2, 0, 1)
    return U, Fh, Y


# speedup vs baseline: 1.0187x; 1.0187x over previous
import functools

import jax
import jax.numpy as jnp
from jax import lax
from jax.experimental import pallas as pl
from jax.experimental.pallas import tpu as pltpu
from jax.experimental.pallas import tpu_sc as plsc

_P = 64
_A = 16
_NC = 2
_NS = 16
_NW = _NC * _NS
_L = 16
_CHUNK = 5120
_NCHUNK = 2
_PC = _P // _NCHUNK
_QR = 128


def _sc_gather_rows(W, tok, k):
    V = W.shape[1]
    N = tok.shape[0]
    mesh = plsc.VectorSubcoreMesh(
        core_axis_name="c", subcore_axis_name="s",
        num_cores=_NC, num_subcores=_NS)

    @functools.partial(
        pl.kernel,
        out_type=jax.ShapeDtypeStruct((_PC, N), jnp.float32),
        mesh=mesh,
        compiler_params=pltpu.CompilerParams(needs_layout_passes=False),
        scratch_types=[
            pltpu.VMEM((N,), jnp.int32),
            pltpu.VMEM((V,), jnp.float32),
            pltpu.VMEM((_CHUNK,), jnp.float32),
            pltpu.VMEM((_CHUNK,), jnp.float32),
            pltpu.SemaphoreType.DMA,
            pltpu.SemaphoreType.DMA,
            pltpu.SemaphoreType.DMA,
            pltpu.SemaphoreType.DMA,
        ],
    )
    def sck(w_hbm, tok_hbm, ut_hbm, tok_v, w_v, out_v0, out_v1,
            sem_t, sem_w, sem_o0, sem_o1):
        wid = lax.axis_index("s") * _NC + lax.axis_index("c")
        ct = pltpu.async_copy(tok_hbm, tok_v, sem_t)
        cw = pltpu.async_copy(w_hbm.at[_PC * k + wid], w_v, sem_w)
        ct.wait()
        cw.wait()

        bufs = (out_v0, out_v1)
        sems = (sem_o0, sem_o1)
        pending = [None, None]
        for c in range(N // _CHUNK):
            buf, sem = bufs[c % 2], sems[c % 2]
            if pending[c % 2] is not None:
                pending[c % 2].wait()
            base = c * _CHUNK

            def g(i, _, buf=buf, base=base):
                idx = tok_v[pl.ds(base + i * _L, _L)]
                buf[pl.ds(i * _L, _L)] = plsc.load_gather(w_v, [idx])
                return 0

            lax.fori_loop(0, _CHUNK // _L, g, 0, unroll=8)
            pending[c % 2] = pltpu.async_copy(
                buf, ut_hbm.at[wid, pl.ds(base, _CHUNK)], sem)
        for h in pending:
            if h is not None:
                h.wait()

    return sck(W, tok)


def _tc_expand_chunk(UTc, SPc, fT, k, prev):
    N = UTc.shape[1]
    PA = _P * _A

    def body(ut_ref, sp_ref, ft_ref, *rest):
        u_ref, fh_ref, y_ref = rest[-3:]
        q = pl.program_id(0)
        ut = ut_ref[...]
        fh = (ut >= sp_ref[...]).astype(jnp.float32)
        t = fh * ut
        ftv = ft_ref[...]
        tiled = jnp.concatenate([ftv] * (_QR // _A), axis=0)
        rloc = (lax.broadcasted_iota(jnp.int32, (_QR, _PC), 0) + q * _QR) // _A
        ploc = lax.broadcasted_iota(jnp.int32, (_QR, _PC), 1)
        et_q = jnp.where(rloc == ploc, tiled, 0.0)
        y_ref[...] = jnp.dot(et_q, t, preferred_element_type=jnp.float32)
        u_ref[...] = ut
        fh_ref[...] = fh

    grid_len = (_PC * _A) // _QR
    hbm = pl.BlockSpec(memory_space=pltpu.MemorySpace.HBM)

    in_specs = [
        pl.BlockSpec((_PC, N), lambda q: (0, 0)),
        pl.BlockSpec((_PC, 1), lambda q: (k, 0)),
        pl.BlockSpec((_A, _PC), lambda q: (0, 0)),
    ]
    operands = [UTc, SPc, fT[:, _PC * k:_PC * (k + 1)]]
    aliases = {}
    if prev is not None:
        in_specs += [hbm, hbm, hbm]
        operands += list(prev)
        aliases = {3: 0, 4: 1, 5: 2}

    return pl.pallas_call(
        body,
        grid=(grid_len,),
        in_specs=in_specs,
        out_specs=[
            pl.BlockSpec((_PC, N), lambda q: (k, 0)),
            pl.BlockSpec((_PC, N), lambda q: (k, 0)),
            pl.BlockSpec((_QR, N), lambda q: (q + grid_len * k, 0)),
        ],
        out_shape=[
            jax.ShapeDtypeStruct((_P, N), jnp.float32),
            jax.ShapeDtypeStruct((_P, N), jnp.float32),
            jax.ShapeDtypeStruct((PA, N), jnp.float32),
        ],
        input_output_aliases=aliases,
    )(*operands)


def kernel(input_ids, W, f, SP):
    B, T = input_ids.shape
    N = B * T
    PA = _P * _A
    tok = input_ids.reshape(N)
    SPc = SP.reshape(_P, 1)
    fT = f.T

    prev = None
    for k in range(_NCHUNK):
        UTc = _sc_gather_rows(W, tok, k)
        prev = _tc_expand_chunk(UTc, SPc, fT, k, prev)
    U_T, FhT, Y2T = prev

    U = U_T.T
    Fh = FhT.T
    Y = Y2T.reshape(_P, _A, N).transpose(2, 0, 1)
    return U, Fh, Y


# speedup vs baseline: 1.1242x; 1.1036x over previous
import functools

import jax
import jax.numpy as jnp
from jax import lax
from jax.experimental import pallas as pl
from jax.experimental.pallas import tpu as pltpu
from jax.experimental.pallas import tpu_sc as plsc

_P = 64
_A = 16
_NC = 2
_NS = 16
_NW = _NC * _NS
_RPW = _P // _NW
_L = 16
_CHUNK = 5120
_QR = 128


def _sc_gather(W, tok):
    V = W.shape[1]
    N = tok.shape[0]
    mesh = plsc.VectorSubcoreMesh(
        core_axis_name="c", subcore_axis_name="s",
        num_cores=_NC, num_subcores=_NS)

    @functools.partial(
        pl.kernel,
        out_type=jax.ShapeDtypeStruct((_P, N), jnp.float32),
        mesh=mesh,
        compiler_params=pltpu.CompilerParams(needs_layout_passes=False),
        scratch_types=[
            pltpu.VMEM((N,), jnp.int32),
            pltpu.VMEM((V,), jnp.float32),
            pltpu.VMEM((_CHUNK,), jnp.float32),
            pltpu.VMEM((_CHUNK,), jnp.float32),
            pltpu.SemaphoreType.DMA,
            pltpu.SemaphoreType.DMA,
            pltpu.SemaphoreType.DMA,
            pltpu.SemaphoreType.DMA,
        ],
    )
    def sck(w_hbm, tok_hbm, ut_hbm, tok_v, w_v, out_v0, out_v1,
            sem_t, sem_w, sem_o0, sem_o1):
        wid = lax.axis_index("s") * _NC + lax.axis_index("c")
        ct = pltpu.async_copy(tok_hbm, tok_v, sem_t)
        bufs = (out_v0, out_v1)
        sems = (sem_o0, sem_o1)
        pending = [None, None]
        for r in range(_RPW):
            p = wid * _RPW + r
            cw = pltpu.async_copy(w_hbm.at[p], w_v, sem_w)
            if r == 0:
                ct.wait()
            cw.wait()
            for c in range(N // _CHUNK):
                j = (r * (N // _CHUNK) + c) % 2
                buf, sem = bufs[j], sems[j]
                if pending[j] is not None:
                    pending[j].wait()
                base = c * _CHUNK

                def g(i, _, buf=buf, base=base):
                    idx = tok_v[pl.ds(base + i * _L, _L)]
                    buf[pl.ds(i * _L, _L)] = plsc.load_gather(w_v, [idx])
                    return 0

                lax.fori_loop(0, _CHUNK // _L, g, 0, unroll=8)
                pending[j] = pltpu.async_copy(
                    buf, ut_hbm.at[p, pl.ds(base, _CHUNK)], sem)
        for h in pending:
            if h is not None:
                h.wait()

    return sck(W, tok)


def _tc_expand(UT, SPc, fT):
    N = UT.shape[1]
    PA = _P * _A

    def body(ut_ref, sp_ref, ft_ref, fh_ref, y_ref):
        q = pl.program_id(0)
        ut = ut_ref[...]
        fh = (ut >= sp_ref[...]).astype(jnp.float32)
        t = fh * ut
        ftv = ft_ref[...]
        tiled = jnp.concatenate([ftv] * (_QR // _A), axis=0)
        rr = lax.broadcasted_iota(jnp.int32, (_QR, _P), 0) + q * _QR
        pp = lax.broadcasted_iota(jnp.int32, (_QR, _P), 1)
        et_q = jnp.where(rr // _A == pp, tiled, 0.0)
        y_ref[...] = jnp.dot(et_q, t, preferred_element_type=jnp.float32)
        fh_ref[...] = fh

    return pl.pallas_call(
        body,
        grid=(PA // _QR,),
        in_specs=[
            pl.BlockSpec((_P, N), lambda q: (0, 0)),
            pl.BlockSpec((_P, 1), lambda q: (0, 0)),
            pl.BlockSpec((_A, _P), lambda q: (0, 0)),
        ],
        out_specs=[
            pl.BlockSpec((_P, N), lambda q: (0, 0)),
            pl.BlockSpec((_QR, N), lambda q: (q, 0)),
        ],
        out_shape=[
            jax.ShapeDtypeStruct((_P, N), jnp.float32),
            jax.ShapeDtypeStruct((PA, N), jnp.float32),
        ],
    )(UT, SPc, fT)


def kernel(input_ids, W, f, SP):
    B, T = input_ids.shape
    N = B * T
    tok = input_ids.reshape(N)
    UT = _sc_gather(W, tok)
    FhT, Y2T = _tc_expand(UT, SP.reshape(_P, 1), f.T)
    U = UT.T
    Fh = FhT.T
    Y = Y2T.reshape(_P, _A, N).transpose(2, 0, 1)
    return U, Fh, Y
